# baseline (speedup 1.0000x reference)
import functools

import jax
import jax.numpy as jnp
from jax import lax
from jax.experimental import pallas as pl
from jax.experimental.pallas import tpu as pltpu
from jax.experimental.pallas import tpu_sc as plsc

NC = 2
NS = 16
NW = NC * NS
CH = 256
CH2 = 512
CH3 = 1024
EA_FRAC_NUM = 16
EA_FRAC_DEN = 25


def _sc_segment_sum(p, src3, dst3, zeros, n_pad):
  F = p.shape[1]
  nchunk = src3.shape[1]
  ch = src3.shape[2]
  rows_per_tile = n_pad // NS
  mesh = plsc.VectorSubcoreMesh(
      core_axis_name="c", subcore_axis_name="s", num_cores=NC,
      num_subcores=NS)

  @functools.partial(
      pl.kernel,
      out_type=jax.ShapeDtypeStruct((2 * n_pad, F), jnp.float32),
      mesh=mesh,
      compiler_params=pltpu.CompilerParams(use_tc_tiling_on_sc=False),
      scratch_types=[
          pltpu.VMEM((nchunk, ch), jnp.int32),
          pltpu.VMEM((nchunk, ch), jnp.int32),
          pltpu.VMEM((ch, F), jnp.float32),
          pltpu.VMEM_SHARED((n_pad, F), jnp.float32),
          pltpu.SemaphoreType.DMA,
      ],
  )
  def seg_kernel(p_hbm, src_hbm, dst_hbm, z_hbm, out_hbm,
                 src_v, dst_v, buf, agg_sh, sem):
    c = lax.axis_index("c")
    s = lax.axis_index("s")
    wid = c * NS + s
    r0 = s * rows_per_tile
    pltpu.sync_copy(src_hbm.at[wid], src_v)
    pltpu.sync_copy(dst_hbm.at[wid], dst_v)
    pltpu.sync_copy(z_hbm.at[pl.ds(r0, rows_per_tile)],
                    agg_sh.at[pl.ds(r0, rows_per_tile)])
    plsc.subcore_barrier()

    def body(j, carry):
      pltpu.async_copy(p_hbm.at[src_v.at[j]], buf, sem).wait()
      pltpu.sync_copy(buf, agg_sh.at[dst_v.at[j]], add=True)
      return carry

    lax.fori_loop(0, nchunk, body, 0)
    plsc.subcore_barrier()
    pltpu.sync_copy(agg_sh.at[pl.ds(r0, rows_per_tile)],
                    out_hbm.at[pl.ds(c * n_pad + r0, rows_per_tile)])

  return seg_kernel(p, src3, dst3, zeros)


def _sc_segment_sum2(p, src2d, dst2d, zeros, n_pad, ch):
  F = p.shape[1]
  e_tile = src2d.shape[1]
  nfull = e_tile // ch
  tail = e_tile - nfull * ch
  rows_per_tile = n_pad // NS
  mesh = plsc.VectorSubcoreMesh(
      core_axis_name="c", subcore_axis_name="s", num_cores=NC,
      num_subcores=NS)

  @functools.partial(
      pl.kernel,
      out_type=jax.ShapeDtypeStruct((2 * n_pad, F), jnp.float32),
      mesh=mesh,
      compiler_params=pltpu.CompilerParams(use_tc_tiling_on_sc=False),
      scratch_types=[
          pltpu.VMEM((e_tile,), jnp.int32),
          pltpu.VMEM((e_tile,), jnp.int32),
          pltpu.VMEM((ch, F), jnp.float32),
          pltpu.VMEM_SHARED((n_pad, F), jnp.float32),
          pltpu.SemaphoreType.DMA,
      ],
  )
  def seg_kernel(p_hbm, src_hbm, dst_hbm, z_hbm, out_hbm,
                 src_v, dst_v, buf, agg_sh, sem):
    c = lax.axis_index("c")
    s = lax.axis_index("s")
    wid = c * NS + s
    r0 = s * rows_per_tile
    pltpu.sync_copy(src_hbm.at[wid], src_v)
    pltpu.sync_copy(dst_hbm.at[wid], dst_v)
    pltpu.sync_copy(z_hbm.at[pl.ds(r0, rows_per_tile)],
                    agg_sh.at[pl.ds(r0, rows_per_tile)])
    plsc.subcore_barrier()

    def body(j, carry):
      pltpu.async_copy(p_hbm.at[src_v.at[pl.ds(j * ch, ch)]], buf, sem).wait()
      pltpu.sync_copy(buf, agg_sh.at[dst_v.at[pl.ds(j * ch, ch)]], add=True)
      return carry

    lax.fori_loop(0, nfull, body, 0)
    if tail:
      o = nfull * ch
      pltpu.async_copy(p_hbm.at[src_v.at[pl.ds(o, tail)]],
                       buf.at[pl.ds(0, tail)], sem).wait()
      pltpu.sync_copy(buf.at[pl.ds(0, tail)],
                      agg_sh.at[dst_v.at[pl.ds(o, tail)]], add=True)
    plsc.subcore_barrier()
    pltpu.sync_copy(agg_sh.at[pl.ds(r0, rows_per_tile)],
                    out_hbm.at[pl.ds(c * n_pad + r0, rows_per_tile)])

  return seg_kernel(p, src2d, dst2d, zeros)


def _sc_segment_sum3(p, srcA, dstA, srcB, dstB, zeros, n_pad, ch):
  F = p.shape[1]
  eA = srcA.shape[1]
  eB = srcB.shape[1]
  e_max = max(eA, eB)
  rows_per_tile = n_pad // NS
  mesh = plsc.VectorSubcoreMesh(
      core_axis_name="c", subcore_axis_name="s", num_cores=NC,
      num_subcores=NS)

  @functools.partial(
      pl.kernel,
      out_type=jax.ShapeDtypeStruct((2 * n_pad, F), jnp.float32),
      mesh=mesh,
      compiler_params=pltpu.CompilerParams(use_tc_tiling_on_sc=False),
      scratch_types=[
          pltpu.VMEM((e_max,), jnp.int32),
          pltpu.VMEM((e_max,), jnp.int32),
          pltpu.VMEM((ch, F), jnp.float32),
          pltpu.VMEM_SHARED((n_pad, F), jnp.float32),
          pltpu.SemaphoreType.DMA,
      ],
  )
  def seg_kernel(p_hbm, srcA_hbm, dstA_hbm, srcB_hbm, dstB_hbm, z_hbm,
                 out_hbm, src_v, dst_v, buf, agg_sh, sem):
    c = lax.axis_index("c")
    s = lax.axis_index("s")
    r0 = s * rows_per_tile
    pltpu.sync_copy(z_hbm.at[pl.ds(r0, rows_per_tile)],
                    agg_sh.at[pl.ds(r0, rows_per_tile)])

    def run(src_hbm, dst_hbm, e_mine):
      pltpu.sync_copy(src_hbm.at[s], src_v.at[pl.ds(0, e_mine)])
      pltpu.sync_copy(dst_hbm.at[s], dst_v.at[pl.ds(0, e_mine)])
      plsc.subcore_barrier()
      nfull = e_mine // ch
      tail = e_mine - nfull * ch

      def body(j, carry):
        pltpu.async_copy(p_hbm.at[src_v.at[pl.ds(j * ch, ch)]], buf,
                         sem).wait()
        pltpu.sync_copy(buf, agg_sh.at[dst_v.at[pl.ds(j * ch, ch)]],
                        add=True)
        return carry

      lax.fori_loop(0, nfull, body, 0)
      if tail:
        o = nfull * ch
        pltpu.async_copy(p_hbm.at[src_v.at[pl.ds(o, tail)]],
                         buf.at[pl.ds(0, tail)], sem).wait()
        pltpu.sync_copy(buf.at[pl.ds(0, tail)],
                        agg_sh.at[dst_v.at[pl.ds(o, tail)]], add=True)
      plsc.subcore_barrier()

    @pl.when(c == 0)
    def _():
      run(srcA_hbm, dstA_hbm, eA)

    @pl.when(c == 1)
    def _():
      run(srcB_hbm, dstB_hbm, eB)

    pltpu.sync_copy(agg_sh.at[pl.ds(r0, rows_per_tile)],
                    out_hbm.at[pl.ds(c * n_pad + r0, rows_per_tile)])

  return seg_kernel(p, srcA, dstA, srcB, dstB, zeros)


def _sc_segment_sum4(p, src2d, dst2d, zeros, n_pad, ch):
  F = p.shape[1]
  e_tile = src2d.shape[1]
  nfull = e_tile // ch
  tail = e_tile - nfull * ch
  rows_per_tile = n_pad // NS
  mesh = plsc.VectorSubcoreMesh(
      core_axis_name="c", subcore_axis_name="s", num_cores=NC,
      num_subcores=NS)

  @functools.partial(
      pl.kernel,
      out_type=jax.ShapeDtypeStruct((2 * n_pad, F), jnp.float32),
      mesh=mesh,
      compiler_params=pltpu.CompilerParams(use_tc_tiling_on_sc=False),
      scratch_types=[
          pltpu.VMEM((2, ch), jnp.int32),
          pltpu.VMEM((2, ch), jnp.int32),
          pltpu.VMEM((ch, F), jnp.float32),
          pltpu.VMEM_SHARED((n_pad, F), jnp.float32),
          pltpu.SemaphoreType.DMA,
          pltpu.SemaphoreType.DMA,
          pltpu.SemaphoreType.DMA,
      ],
  )
  def seg_kernel(p_hbm, src_hbm, dst_hbm, z_hbm, out_hbm,
                 src_v, dst_v, buf, agg_sh, semg, semi0, semi1):
    c = lax.axis_index("c")
    s = lax.axis_index("s")
    wid = c * NS + s
    r0 = s * rows_per_tile
    semi = (semi0, semi1)
    pltpu.async_copy(src_hbm.at[wid, pl.ds(0, ch)], src_v.at[0], semi0)
    pltpu.async_copy(dst_hbm.at[wid, pl.ds(0, ch)], dst_v.at[0], semi0)
    pltpu.async_copy(src_hbm.at[wid, pl.ds(ch, ch)], src_v.at[1], semi1)
    pltpu.async_copy(dst_hbm.at[wid, pl.ds(ch, ch)], dst_v.at[1], semi1)
    pltpu.sync_copy(z_hbm.at[pl.ds(r0, rows_per_tile)],
                    agg_sh.at[pl.ds(r0, rows_per_tile)])
    plsc.subcore_barrier()

    def body(gg, carry):
      for b in range(2):
        j = 2 * gg + b
        pltpu.make_async_copy(src_hbm.at[wid, pl.ds(0, ch)], src_v.at[b],
                              semi[b]).wait()
        pltpu.make_async_copy(dst_hbm.at[wid, pl.ds(0, ch)], dst_v.at[b],
                              semi[b]).wait()
        pltpu.async_copy(p_hbm.at[src_v.at[b]], buf, semg).wait()
        pltpu.sync_copy(buf, agg_sh.at[dst_v.at[b]], add=True)

        @pl.when(j + 2 < nfull)
        def _():
          o = (j + 2) * ch
          pltpu.async_copy(src_hbm.at[wid, pl.ds(o, ch)], src_v.at[b],
                           semi[b])
          pltpu.async_copy(dst_hbm.at[wid, pl.ds(o, ch)], dst_v.at[b],
                           semi[b])

      return carry

    lax.fori_loop(0, nfull // 2, body, 0)
    if nfull % 2:
      pltpu.make_async_copy(src_hbm.at[wid, pl.ds(0, ch)], src_v.at[0],
                            semi0).wait()
      pltpu.make_async_copy(dst_hbm.at[wid, pl.ds(0, ch)], dst_v.at[0],
                            semi0).wait()
      pltpu.async_copy(p_hbm.at[src_v.at[0]], buf, semg).wait()
      pltpu.sync_copy(buf, agg_sh.at[dst_v.at[0]], add=True)
    if tail:
      o = nfull * ch
      pltpu.sync_copy(src_hbm.at[wid, pl.ds(o, tail)],
                      src_v.at[1, pl.ds(0, tail)])
      pltpu.sync_copy(dst_hbm.at[wid, pl.ds(o, tail)],
                      dst_v.at[1, pl.ds(0, tail)])
      pltpu.async_copy(p_hbm.at[src_v.at[1, pl.ds(0, tail)]],
                       buf.at[pl.ds(0, tail)], semg).wait()
      pltpu.sync_copy(buf.at[pl.ds(0, tail)],
                      agg_sh.at[dst_v.at[1, pl.ds(0, tail)]], add=True)
    plsc.subcore_barrier()
    pltpu.sync_copy(agg_sh.at[pl.ds(r0, rows_per_tile)],
                    out_hbm.at[pl.ds(c * n_pad + r0, rows_per_tile)])

  return seg_kernel(p, src2d, dst2d, zeros)


def _project1(x, wlT, wrT, n_pad, blk):
  D = x.shape[1]
  F1 = wlT.shape[1]
  grid = n_pad // blk

  def body(x_ref, wl_ref, wr_ref, p_ref, q_ref):
    xb = x_ref[...]
    pb = jnp.dot(xb, wl_ref[...], preferred_element_type=jnp.float32)
    q_ref[...] = jnp.dot(xb, wr_ref[...], preferred_element_type=jnp.float32)
    ones = jnp.ones((blk, 1), jnp.float32)
    zer = jnp.zeros((blk, 15), jnp.float32)
    p_ref[...] = jnp.concatenate([pb, ones, zer], axis=1)

  return pl.pallas_call(
      body,
      grid=(grid,),
      in_specs=[
          pl.BlockSpec((blk, D), lambda i: (i, 0)),
          pl.BlockSpec((D, F1), lambda i: (0, 0)),
          pl.BlockSpec((D, F1), lambda i: (0, 0)),
      ],
      out_specs=[
          pl.BlockSpec((blk, F1 + 16), lambda i: (i, 0)),
          pl.BlockSpec((blk, F1), lambda i: (i, 0)),
      ],
      out_shape=[
          jax.ShapeDtypeStruct((n_pad, F1 + 16), jnp.float32),
          jax.ShapeDtypeStruct((n_pad, F1), jnp.float32),
      ],
  )(x, wlT, wrT)


def _combine1(parts, q1, b1, wlT2, wrT2, n_pad, blk):
  Fw = parts.shape[1]
  F1 = Fw - 16
  F2 = wlT2.shape[1]
  grid = n_pad // blk

  def body(pa_ref, pb_ref, q_ref, b_ref, wl_ref, wr_ref,
           p2_ref, q2_ref, inv_ref):
    ps = pa_ref[...] + pb_ref[...]
    cnt = ps[:, F1:F1 + 1]
    inv = 1.0 / jnp.maximum(cnt, 1.0)
    o = ps[:, :F1] * inv + b_ref[...] + q_ref[...]
    nrm = jnp.sqrt(jnp.sum(o * o, axis=1, keepdims=True))
    h = o / jnp.maximum(nrm, 1e-12)
    h = jnp.maximum(h, 0.0)
    p2_ref[...] = jnp.dot(h, wl_ref[...], preferred_element_type=jnp.float32)
    q2_ref[...] = jnp.dot(h, wr_ref[...], preferred_element_type=jnp.float32)
    inv_ref[...] = inv

  return pl.pallas_call(
      body,
      grid=(grid,),
      in_specs=[
          pl.BlockSpec((blk, Fw), lambda i: (i, 0)),
          pl.BlockSpec((blk, Fw), lambda i: (i + grid, 0)),
          pl.BlockSpec((blk, F1), lambda i: (i, 0)),
          pl.BlockSpec((1, F1), lambda i: (0, 0)),
          pl.BlockSpec((F1, F2), lambda i: (0, 0)),
          pl.BlockSpec((F1, F2), lambda i: (0, 0)),
      ],
      out_specs=[
          pl.BlockSpec((blk, F2), lambda i: (i, 0)),
          pl.BlockSpec((blk, F2), lambda i: (i, 0)),
          pl.BlockSpec((blk, 1), lambda i: (i, 0)),
      ],
      out_shape=[
          jax.ShapeDtypeStruct((n_pad, F2), jnp.float32),
          jax.ShapeDtypeStruct((n_pad, F2), jnp.float32),
          jax.ShapeDtypeStruct((n_pad, 1), jnp.float32),
      ],
  )(parts, parts, q1, b1[None, :], wlT2, wrT2)


def _combine2(parts, q2, b2, inv, wlT3, wrT3, n_pad, blk):
  F2 = parts.shape[1]
  F3 = wlT3.shape[1]
  grid = n_pad // blk

  def body(pa_ref, pb_ref, q_ref, b_ref, inv_ref, wl_ref, wr_ref,
           p3_ref, q3_ref):
    ps = pa_ref[...] + pb_ref[...]
    o = ps * inv_ref[...] + b_ref[...] + q_ref[...]
    nrm = jnp.sqrt(jnp.sum(o * o, axis=1, keepdims=True))
    h = o / jnp.maximum(nrm, 1e-12)
    h = jnp.maximum(h, 0.0)
    p3_ref[...] = jnp.dot(h, wl_ref[...], preferred_element_type=jnp.float32)
    q3_ref[...] = jnp.dot(h, wr_ref[...], preferred_element_type=jnp.float32)

  return pl.pallas_call(
      body,
      grid=(grid,),
      in_specs=[
          pl.BlockSpec((blk, F2), lambda i: (i, 0)),
          pl.BlockSpec((blk, F2), lambda i: (i + grid, 0)),
          pl.BlockSpec((blk, F2), lambda i: (i, 0)),
          pl.BlockSpec((1, F2), lambda i: (0, 0)),
          pl.BlockSpec((blk, 1), lambda i: (i, 0)),
          pl.BlockSpec((F2, F3), lambda i: (0, 0)),
          pl.BlockSpec((F2, F3), lambda i: (0, 0)),
      ],
      out_specs=[
          pl.BlockSpec((blk, F3), lambda i: (i, 0)),
          pl.BlockSpec((blk, F3), lambda i: (i, 0)),
      ],
      out_shape=[
          jax.ShapeDtypeStruct((n_pad, F3), jnp.float32),
          jax.ShapeDtypeStruct((n_pad, F3), jnp.float32),
      ],
  )(parts, parts, q2, b2[None, :], inv, wlT3, wrT3)


def _final(parts, q3, b3, inv, watt, wfcT, bfc, wsT, bs, n, n_pad):
  F3 = watt.shape[0]
  HID = wfcT.shape[1]

  def body(pa_ref, pb_ref, q_ref, b_ref, inv_ref, wa_ref, wfc_ref, bfc_ref,
           ws_ref, bs_ref, out_ref):
    ps = pa_ref[...] + pb_ref[...]
    o = ps * inv_ref[...] + b_ref[...] + q_ref[...]
    nrm = jnp.sqrt(jnp.sum(o * o, axis=1, keepdims=True))
    h = o / jnp.maximum(nrm, 1e-12)
    cs = jnp.sum(h, axis=0, keepdims=True) / n
    gc = jnp.dot(cs, wa_ref[...], preferred_element_type=jnp.float32)
    tg = jnp.tanh(gc)
    scores = jax.nn.sigmoid(jnp.sum(h * tg, axis=1, keepdims=True))
    rep = jnp.sum(h * scores, axis=0, keepdims=True)
    s1 = jnp.dot(rep, wfc_ref[...], preferred_element_type=jnp.float32)
    s1 = jnp.maximum(s1 + bfc_ref[...], 0.0)
    s2 = jnp.dot(s1, ws_ref[...], preferred_element_type=jnp.float32)
    out_ref[...] = jax.nn.sigmoid(s2 + bs_ref[...])

  return pl.pallas_call(
      body,
      in_specs=[
          pl.BlockSpec((n, F3), lambda: (0, 0)),
          pl.BlockSpec((n, F3), lambda: (0, 0)),
          pl.BlockSpec((n, F3), lambda: (0, 0)),
          pl.BlockSpec((1, F3), lambda: (0, 0)),
          pl.BlockSpec((n, 1), lambda: (0, 0)),
          pl.BlockSpec((F3, F3), lambda: (0, 0)),
          pl.BlockSpec((F3, HID), lambda: (0, 0)),
          pl.BlockSpec((1, HID), lambda: (0, 0)),
          pl.BlockSpec((HID, 1), lambda: (0, 0)),
          pl.BlockSpec((1, 1), lambda: (0, 0)),
      ],
      out_specs=pl.BlockSpec((1, 1), lambda: (0, 0)),
      out_shape=jax.ShapeDtypeStruct((1, 1), jnp.float32),
  )(parts[:n], parts[n_pad:n_pad + n], q3[:n], b3[None, :], inv[:n],
    watt, wfcT, bfc[None, :], wsT, bs[None, :])


def kernel(features_1, edge_index_1, W_l1, b_l1, W_r1, W_l2, b_l2, W_r2,
           W_l3, b_l3, W_r3, W_att, W_fc, b_fc, W_s, b_s):
  n, d = features_1.shape
  e = edge_index_1.shape[1]

  n_pad = ((n + NS * 8 - 1) // (NS * 8)) * (NS * 8)
  e_tile = ((e + NW * CH - 1) // (NW * CH)) * CH
  e_pad = e_tile * NW

  f1 = W_l1.shape[0]
  f2 = W_l2.shape[0]
  f3 = W_l3.shape[0]

  x = jnp.zeros((n_pad, d), jnp.float32).at[:n].set(features_1)
  src = jnp.concatenate(
      [edge_index_1[0].astype(jnp.int32),
       jnp.zeros((e_pad - e,), jnp.int32)]).reshape(NW, e_tile)
  pad_dst = n + jnp.arange(e_pad - e, dtype=jnp.int32) % (n_pad - n)
  dst = jnp.concatenate(
      [edge_index_1[1].astype(jnp.int32), pad_dst]).reshape(NW, e_tile)
  del pad_dst

  z1 = jnp.zeros((n_pad, f1 + 16), jnp.float32)
  z2 = jnp.zeros((n_pad, f2), jnp.float32)
  z3 = jnp.zeros((n_pad, f3), jnp.float32)

  blk = n_pad // 8

  p1, q1 = _project1(x, W_l1.T, W_r1.T, n_pad, blk)
  parts1 = _sc_segment_sum4(p1, src, dst, z1, n_pad, CH)
  p2, q2, inv = _combine1(parts1, q1, b_l1, W_l2.T, W_r2.T, n_pad, blk)
  parts2 = _sc_segment_sum2(p2, src, dst, z2, n_pad, CH2)
  p3, q3 = _combine2(parts2, q2, b_l2, inv, W_l3.T, W_r3.T, n_pad, blk)
  parts3 = _sc_segment_sum2(p3, src, dst, z3, n_pad, CH3)
  return _final(parts3, q3, b_l3, inv, W_att, W_fc.T, b_fc, W_s.T, b_s,
                n, n_pad)


# speedup vs baseline: 1.4041x; 1.4041x over previous
import functools

import jax
import jax.numpy as jnp
from jax import lax
from jax.experimental import pallas as pl
from jax.experimental.pallas import tpu as pltpu
from jax.experimental.pallas import tpu_sc as plsc

NC = 2
NS = 16
NW = NC * NS
CH = 128
CH2 = 512
CH3 = 1024
EA_FRAC_NUM = 16
EA_FRAC_DEN = 25


def _sc_segment_sum(p, src3, dst3, zeros, n_pad):
  F = p.shape[1]
  nchunk = src3.shape[1]
  ch = src3.shape[2]
  rows_per_tile = n_pad // NS
  mesh = plsc.VectorSubcoreMesh(
      core_axis_name="c", subcore_axis_name="s", num_cores=NC,
      num_subcores=NS)

  @functools.partial(
      pl.kernel,
      out_type=jax.ShapeDtypeStruct((2 * n_pad, F), jnp.float32),
      mesh=mesh,
      compiler_params=pltpu.CompilerParams(use_tc_tiling_on_sc=False),
      scratch_types=[
          pltpu.VMEM((nchunk, ch), jnp.int32),
          pltpu.VMEM((nchunk, ch), jnp.int32),
          pltpu.VMEM((ch, F), jnp.float32),
          pltpu.VMEM_SHARED((n_pad, F), jnp.float32),
          pltpu.SemaphoreType.DMA,
      ],
  )
  def seg_kernel(p_hbm, src_hbm, dst_hbm, z_hbm, out_hbm,
                 src_v, dst_v, buf, agg_sh, sem):
    c = lax.axis_index("c")
    s = lax.axis_index("s")
    wid = c * NS + s
    r0 = s * rows_per_tile
    pltpu.sync_copy(src_hbm.at[wid], src_v)
    pltpu.sync_copy(dst_hbm.at[wid], dst_v)
    pltpu.sync_copy(z_hbm.at[pl.ds(r0, rows_per_tile)],
                    agg_sh.at[pl.ds(r0, rows_per_tile)])
    plsc.subcore_barrier()

    def body(j, carry):
      pltpu.async_copy(p_hbm.at[src_v.at[j]], buf, sem).wait()
      pltpu.sync_copy(buf, agg_sh.at[dst_v.at[j]], add=True)
      return carry

    lax.fori_loop(0, nchunk, body, 0)
    plsc.subcore_barrier()
    pltpu.sync_copy(agg_sh.at[pl.ds(r0, rows_per_tile)],
                    out_hbm.at[pl.ds(c * n_pad + r0, rows_per_tile)])

  return seg_kernel(p, src3, dst3, zeros)


def _sc_segment_sum2(p, src2d, dst2d, zeros, n_pad, ch):
  F = p.shape[1]
  e_tile = src2d.shape[1]
  nfull = e_tile // ch
  tail = e_tile - nfull * ch
  rows_per_tile = n_pad // NS
  mesh = plsc.VectorSubcoreMesh(
      core_axis_name="c", subcore_axis_name="s", num_cores=NC,
      num_subcores=NS)

  @functools.partial(
      pl.kernel,
      out_type=jax.ShapeDtypeStruct((2 * n_pad, F), jnp.float32),
      mesh=mesh,
      compiler_params=pltpu.CompilerParams(use_tc_tiling_on_sc=False),
      scratch_types=[
          pltpu.VMEM((e_tile,), jnp.int32),
          pltpu.VMEM((e_tile,), jnp.int32),
          pltpu.VMEM((ch, F), jnp.float32),
          pltpu.VMEM_SHARED((n_pad, F), jnp.float32),
          pltpu.SemaphoreType.DMA,
      ],
  )
  def seg_kernel(p_hbm, src_hbm, dst_hbm, z_hbm, out_hbm,
                 src_v, dst_v, buf, agg_sh, sem):
    c = lax.axis_index("c")
    s = lax.axis_index("s")
    wid = c * NS + s
    r0 = s * rows_per_tile
    pltpu.sync_copy(src_hbm.at[wid], src_v)
    pltpu.sync_copy(dst_hbm.at[wid], dst_v)
    pltpu.sync_copy(z_hbm.at[pl.ds(r0, rows_per_tile)],
                    agg_sh.at[pl.ds(r0, rows_per_tile)])
    plsc.subcore_barrier()

    def body(j, carry):
      pltpu.async_copy(p_hbm.at[src_v.at[pl.ds(j * ch, ch)]], buf, sem).wait()
      pltpu.sync_copy(buf, agg_sh.at[dst_v.at[pl.ds(j * ch, ch)]], add=True)
      return carry

    lax.fori_loop(0, nfull, body, 0)
    if tail:
      o = nfull * ch
      pltpu.async_copy(p_hbm.at[src_v.at[pl.ds(o, tail)]],
                       buf.at[pl.ds(0, tail)], sem).wait()
      pltpu.sync_copy(buf.at[pl.ds(0, tail)],
                      agg_sh.at[dst_v.at[pl.ds(o, tail)]], add=True)
    plsc.subcore_barrier()
    pltpu.sync_copy(agg_sh.at[pl.ds(r0, rows_per_tile)],
                    out_hbm.at[pl.ds(c * n_pad + r0, rows_per_tile)])

  return seg_kernel(p, src2d, dst2d, zeros)


def _sc_segment_sum3(p, srcA, dstA, srcB, dstB, zeros, n_pad, ch):
  F = p.shape[1]
  eA = srcA.shape[1]
  eB = srcB.shape[1]
  e_max = max(eA, eB)
  rows_per_tile = n_pad // NS
  mesh = plsc.VectorSubcoreMesh(
      core_axis_name="c", subcore_axis_name="s", num_cores=NC,
      num_subcores=NS)

  @functools.partial(
      pl.kernel,
      out_type=jax.ShapeDtypeStruct((2 * n_pad, F), jnp.float32),
      mesh=mesh,
      compiler_params=pltpu.CompilerParams(use_tc_tiling_on_sc=False),
      scratch_types=[
          pltpu.VMEM((e_max,), jnp.int32),
          pltpu.VMEM((e_max,), jnp.int32),
          pltpu.VMEM((ch, F), jnp.float32),
          pltpu.VMEM_SHARED((n_pad, F), jnp.float32),
          pltpu.SemaphoreType.DMA,
      ],
  )
  def seg_kernel(p_hbm, srcA_hbm, dstA_hbm, srcB_hbm, dstB_hbm, z_hbm,
                 out_hbm, src_v, dst_v, buf, agg_sh, sem):
    c = lax.axis_index("c")
    s = lax.axis_index("s")
    r0 = s * rows_per_tile
    pltpu.sync_copy(z_hbm.at[pl.ds(r0, rows_per_tile)],
                    agg_sh.at[pl.ds(r0, rows_per_tile)])

    def run(src_hbm, dst_hbm, e_mine):
      pltpu.sync_copy(src_hbm.at[s], src_v.at[pl.ds(0, e_mine)])
      pltpu.sync_copy(dst_hbm.at[s], dst_v.at[pl.ds(0, e_mine)])
      plsc.subcore_barrier()
      nfull = e_mine // ch
      tail = e_mine - nfull * ch

      def body(j, carry):
        pltpu.async_copy(p_hbm.at[src_v.at[pl.ds(j * ch, ch)]], buf,
                         sem).wait()
        pltpu.sync_copy(buf, agg_sh.at[dst_v.at[pl.ds(j * ch, ch)]],
                        add=True)
        return carry

      lax.fori_loop(0, nfull, body, 0)
      if tail:
        o = nfull * ch
        pltpu.async_copy(p_hbm.at[src_v.at[pl.ds(o, tail)]],
                         buf.at[pl.ds(0, tail)], sem).wait()
        pltpu.sync_copy(buf.at[pl.ds(0, tail)],
                        agg_sh.at[dst_v.at[pl.ds(o, tail)]], add=True)
      plsc.subcore_barrier()

    @pl.when(c == 0)
    def _():
      run(srcA_hbm, dstA_hbm, eA)

    @pl.when(c == 1)
    def _():
      run(srcB_hbm, dstB_hbm, eB)

    pltpu.sync_copy(agg_sh.at[pl.ds(r0, rows_per_tile)],
                    out_hbm.at[pl.ds(c * n_pad + r0, rows_per_tile)])

  return seg_kernel(p, srcA, dstA, srcB, dstB, zeros)


def _sc_segment_sum4(p, src2d, dst2d, zeros, n_pad, ch):
  F = p.shape[1]
  e_tile = src2d.shape[1]
  nfull = e_tile // ch
  tail = e_tile - nfull * ch
  rows_per_tile = n_pad // NS
  mesh = plsc.VectorSubcoreMesh(
      core_axis_name="c", subcore_axis_name="s", num_cores=NC,
      num_subcores=NS)

  @functools.partial(
      pl.kernel,
      out_type=jax.ShapeDtypeStruct((2 * n_pad, F), jnp.float32),
      mesh=mesh,
      compiler_params=pltpu.CompilerParams(use_tc_tiling_on_sc=False),
      scratch_types=[
          pltpu.VMEM((2, ch), jnp.int32),
          pltpu.VMEM((2, ch), jnp.int32),
          pltpu.VMEM((ch, F), jnp.float32),
          pltpu.VMEM_SHARED((n_pad, F), jnp.float32),
          pltpu.SemaphoreType.DMA,
          pltpu.SemaphoreType.DMA,
          pltpu.SemaphoreType.DMA,
      ],
  )
  def seg_kernel(p_hbm, src_hbm, dst_hbm, z_hbm, out_hbm,
                 src_v, dst_v, buf, agg_sh, semg, semi0, semi1):
    c = lax.axis_index("c")
    s = lax.axis_index("s")
    wid = c * NS + s
    r0 = s * rows_per_tile
    semi = (semi0, semi1)
    pltpu.async_copy(src_hbm.at[wid, pl.ds(0, ch)], src_v.at[0], semi0)
    pltpu.async_copy(dst_hbm.at[wid, pl.ds(0, ch)], dst_v.at[0], semi0)
    pltpu.async_copy(src_hbm.at[wid, pl.ds(ch, ch)], src_v.at[1], semi1)
    pltpu.async_copy(dst_hbm.at[wid, pl.ds(ch, ch)], dst_v.at[1], semi1)
    pltpu.sync_copy(z_hbm.at[pl.ds(r0, rows_per_tile)],
                    agg_sh.at[pl.ds(r0, rows_per_tile)])
    plsc.subcore_barrier()

    def body(gg, carry):
      for b in range(2):
        j = 2 * gg + b
        pltpu.make_async_copy(src_hbm.at[wid, pl.ds(0, ch)], src_v.at[b],
                              semi[b]).wait()
        pltpu.make_async_copy(dst_hbm.at[wid, pl.ds(0, ch)], dst_v.at[b],
                              semi[b]).wait()
        pltpu.async_copy(p_hbm.at[src_v.at[b]], buf, semg).wait()
        pltpu.sync_copy(buf, agg_sh.at[dst_v.at[b]], add=True)

        @pl.when(j + 2 < nfull)
        def _():
          o = (j + 2) * ch
          pltpu.async_copy(src_hbm.at[wid, pl.ds(o, ch)], src_v.at[b],
                           semi[b])
          pltpu.async_copy(dst_hbm.at[wid, pl.ds(o, ch)], dst_v.at[b],
                           semi[b])

      return carry

    lax.fori_loop(0, nfull // 2, body, 0)
    if nfull % 2:
      pltpu.make_async_copy(src_hbm.at[wid, pl.ds(0, ch)], src_v.at[0],
                            semi0).wait()
      pltpu.make_async_copy(dst_hbm.at[wid, pl.ds(0, ch)], dst_v.at[0],
                            semi0).wait()
      pltpu.async_copy(p_hbm.at[src_v.at[0]], buf, semg).wait()
      pltpu.sync_copy(buf, agg_sh.at[dst_v.at[0]], add=True)
    if tail:
      o = nfull * ch
      pltpu.sync_copy(src_hbm.at[wid, pl.ds(o, tail)],
                      src_v.at[1, pl.ds(0, tail)])
      pltpu.sync_copy(dst_hbm.at[wid, pl.ds(o, tail)],
                      dst_v.at[1, pl.ds(0, tail)])
      pltpu.async_copy(p_hbm.at[src_v.at[1, pl.ds(0, tail)]],
                       buf.at[pl.ds(0, tail)], semg).wait()
      pltpu.sync_copy(buf.at[pl.ds(0, tail)],
                      agg_sh.at[dst_v.at[1, pl.ds(0, tail)]], add=True)
    plsc.subcore_barrier()
    pltpu.sync_copy(agg_sh.at[pl.ds(r0, rows_per_tile)],
                    out_hbm.at[pl.ds(c * n_pad + r0, rows_per_tile)])

  return seg_kernel(p, src2d, dst2d, zeros)


def _project1(x, wlT, wrT, n_pad, blk):
  D = x.shape[1]
  F1 = wlT.shape[1]
  grid = n_pad // blk

  def body(x_ref, wl_ref, wr_ref, p_ref, q_ref):
    xb = x_ref[...]
    pb = jnp.dot(xb, wl_ref[...], preferred_element_type=jnp.float32)
    q_ref[...] = jnp.dot(xb, wr_ref[...], preferred_element_type=jnp.float32)
    ones = jnp.ones((blk, 1), jnp.float32)
    zer = jnp.zeros((blk, 15), jnp.float32)
    p_ref[...] = jnp.concatenate([pb, ones, zer], axis=1)

  return pl.pallas_call(
      body,
      grid=(grid,),
      in_specs=[
          pl.BlockSpec((blk, D), lambda i: (i, 0)),
          pl.BlockSpec((D, F1), lambda i: (0, 0)),
          pl.BlockSpec((D, F1), lambda i: (0, 0)),
      ],
      out_specs=[
          pl.BlockSpec((blk, F1 + 16), lambda i: (i, 0)),
          pl.BlockSpec((blk, F1), lambda i: (i, 0)),
      ],
      out_shape=[
          jax.ShapeDtypeStruct((n_pad, F1 + 16), jnp.float32),
          jax.ShapeDtypeStruct((n_pad, F1), jnp.float32),
      ],
  )(x, wlT, wrT)


def _combine1(parts, q1, b1, wlT2, wrT2, n_pad, blk):
  Fw = parts.shape[1]
  F1 = Fw - 16
  F2 = wlT2.shape[1]
  grid = n_pad // blk

  def body(pa_ref, pb_ref, q_ref, b_ref, wl_ref, wr_ref,
           p2_ref, q2_ref, inv_ref):
    ps = pa_ref[...] + pb_ref[...]
    cnt = ps[:, F1:F1 + 1]
    inv = 1.0 / jnp.maximum(cnt, 1.0)
    o = ps[:, :F1] * inv + b_ref[...] + q_ref[...]
    nrm = jnp.sqrt(jnp.sum(o * o, axis=1, keepdims=True))
    h = o / jnp.maximum(nrm, 1e-12)
    h = jnp.maximum(h, 0.0)
    p2_ref[...] = jnp.dot(h, wl_ref[...], preferred_element_type=jnp.float32)
    q2_ref[...] = jnp.dot(h, wr_ref[...], preferred_element_type=jnp.float32)
    inv_ref[...] = inv

  return pl.pallas_call(
      body,
      grid=(grid,),
      in_specs=[
          pl.BlockSpec((blk, Fw), lambda i: (i, 0)),
          pl.BlockSpec((blk, Fw), lambda i: (i + grid, 0)),
          pl.BlockSpec((blk, F1), lambda i: (i, 0)),
          pl.BlockSpec((1, F1), lambda i: (0, 0)),
          pl.BlockSpec((F1, F2), lambda i: (0, 0)),
          pl.BlockSpec((F1, F2), lambda i: (0, 0)),
      ],
      out_specs=[
          pl.BlockSpec((blk, F2), lambda i: (i, 0)),
          pl.BlockSpec((blk, F2), lambda i: (i, 0)),
          pl.BlockSpec((blk, 1), lambda i: (i, 0)),
      ],
      out_shape=[
          jax.ShapeDtypeStruct((n_pad, F2), jnp.float32),
          jax.ShapeDtypeStruct((n_pad, F2), jnp.float32),
          jax.ShapeDtypeStruct((n_pad, 1), jnp.float32),
      ],
  )(parts, parts, q1, b1[None, :], wlT2, wrT2)


def _combine2(parts, q2, b2, inv, wlT3, wrT3, n_pad, blk):
  F2 = parts.shape[1]
  F3 = wlT3.shape[1]
  grid = n_pad // blk

  def body(pa_ref, pb_ref, q_ref, b_ref, inv_ref, wl_ref, wr_ref,
           p3_ref, q3_ref):
    ps = pa_ref[...] + pb_ref[...]
    o = ps * inv_ref[...] + b_ref[...] + q_ref[...]
    nrm = jnp.sqrt(jnp.sum(o * o, axis=1, keepdims=True))
    h = o / jnp.maximum(nrm, 1e-12)
    h = jnp.maximum(h, 0.0)
    p3_ref[...] = jnp.dot(h, wl_ref[...], preferred_element_type=jnp.float32)
    q3_ref[...] = jnp.dot(h, wr_ref[...], preferred_element_type=jnp.float32)

  return pl.pallas_call(
      body,
      grid=(grid,),
      in_specs=[
          pl.BlockSpec((blk, F2), lambda i: (i, 0)),
          pl.BlockSpec((blk, F2), lambda i: (i + grid, 0)),
          pl.BlockSpec((blk, F2), lambda i: (i, 0)),
          pl.BlockSpec((1, F2), lambda i: (0, 0)),
          pl.BlockSpec((blk, 1), lambda i: (i, 0)),
          pl.BlockSpec((F2, F3), lambda i: (0, 0)),
          pl.BlockSpec((F2, F3), lambda i: (0, 0)),
      ],
      out_specs=[
          pl.BlockSpec((blk, F3), lambda i: (i, 0)),
          pl.BlockSpec((blk, F3), lambda i: (i, 0)),
      ],
      out_shape=[
          jax.ShapeDtypeStruct((n_pad, F3), jnp.float32),
          jax.ShapeDtypeStruct((n_pad, F3), jnp.float32),
      ],
  )(parts, parts, q2, b2[None, :], inv, wlT3, wrT3)


def _final(parts, q3, b3, inv, watt, wfcT, bfc, wsT, bs, n, n_pad):
  F3 = watt.shape[0]
  HID = wfcT.shape[1]

  def body(pa_ref, pb_ref, q_ref, b_ref, inv_ref, wa_ref, wfc_ref, bfc_ref,
           ws_ref, bs_ref, out_ref):
    ps = pa_ref[...] + pb_ref[...]
    o = ps * inv_ref[...] + b_ref[...] + q_ref[...]
    nrm = jnp.sqrt(jnp.sum(o * o, axis=1, keepdims=True))
    h = o / jnp.maximum(nrm, 1e-12)
    cs = jnp.sum(h, axis=0, keepdims=True) / n
    gc = jnp.dot(cs, wa_ref[...], preferred_element_type=jnp.float32)
    tg = jnp.tanh(gc)
    scores = jax.nn.sigmoid(jnp.sum(h * tg, axis=1, keepdims=True))
    rep = jnp.sum(h * scores, axis=0, keepdims=True)
    s1 = jnp.dot(rep, wfc_ref[...], preferred_element_type=jnp.float32)
    s1 = jnp.maximum(s1 + bfc_ref[...], 0.0)
    s2 = jnp.dot(s1, ws_ref[...], preferred_element_type=jnp.float32)
    out_ref[...] = jax.nn.sigmoid(s2 + bs_ref[...])

  return pl.pallas_call(
      body,
      in_specs=[
          pl.BlockSpec((n, F3), lambda: (0, 0)),
          pl.BlockSpec((n, F3), lambda: (0, 0)),
          pl.BlockSpec((n, F3), lambda: (0, 0)),
          pl.BlockSpec((1, F3), lambda: (0, 0)),
          pl.BlockSpec((n, 1), lambda: (0, 0)),
          pl.BlockSpec((F3, F3), lambda: (0, 0)),
          pl.BlockSpec((F3, HID), lambda: (0, 0)),
          pl.BlockSpec((1, HID), lambda: (0, 0)),
          pl.BlockSpec((HID, 1), lambda: (0, 0)),
          pl.BlockSpec((1, 1), lambda: (0, 0)),
      ],
      out_specs=pl.BlockSpec((1, 1), lambda: (0, 0)),
      out_shape=jax.ShapeDtypeStruct((1, 1), jnp.float32),
  )(parts[:n], parts[n_pad:n_pad + n], q3[:n], b3[None, :], inv[:n],
    watt, wfcT, bfc[None, :], wsT, bs[None, :])


def kernel(features_1, edge_index_1, W_l1, b_l1, W_r1, W_l2, b_l2, W_r2,
           W_l3, b_l3, W_r3, W_att, W_fc, b_fc, W_s, b_s):
  n, d = features_1.shape
  e = edge_index_1.shape[1]

  n_pad = ((n + NS * 8 - 1) // (NS * 8)) * (NS * 8)
  e_tile = ((e + NW * CH - 1) // (NW * CH)) * CH
  e_pad = e_tile * NW

  f1 = W_l1.shape[0]
  f2 = W_l2.shape[0]
  f3 = W_l3.shape[0]

  x = jnp.zeros((n_pad, d), jnp.float32).at[:n].set(features_1)
  src = jnp.concatenate(
      [edge_index_1[0].astype(jnp.int32),
       jnp.zeros((e_pad - e,), jnp.int32)]).reshape(NW, e_tile)
  pad_dst = n + jnp.arange(e_pad - e, dtype=jnp.int32) % (n_pad - n)
  dst = jnp.concatenate(
      [edge_index_1[1].astype(jnp.int32), pad_dst]).reshape(NW, e_tile)
  del pad_dst

  z1 = jnp.zeros((n_pad, f1 + 16), jnp.float32)
  z2 = jnp.zeros((n_pad, f2), jnp.float32)
  z3 = jnp.zeros((n_pad, f3), jnp.float32)

  blk = n_pad // 8

  p1, q1 = _project1(x, W_l1.T, W_r1.T, n_pad, blk)
  parts1 = _sc_segment_sum2(p1, src, dst, z1, n_pad, CH)
  p2, q2, inv = _combine1(parts1, q1, b_l1, W_l2.T, W_r2.T, n_pad, blk)
  parts2 = _sc_segment_sum2(p2, src, dst, z2, n_pad, CH2)
  p3, q3 = _combine2(parts2, q2, b_l2, inv, W_l3.T, W_r3.T, n_pad, blk)
  parts3 = _sc_segment_sum2(p3, src, dst, z3, n_pad, CH3)
  return _final(parts3, q3, b_l3, inv, W_att, W_fc.T, b_fc, W_s.T, b_s,
                n, n_pad)


# speedup vs baseline: 1.4051x; 1.0007x over previous
import functools

import jax
import jax.numpy as jnp
from jax import lax
from jax.experimental import pallas as pl
from jax.experimental.pallas import tpu as pltpu
from jax.experimental.pallas import tpu_sc as plsc

NC = 2
NS = 16
NW = NC * NS
CH = 128
CH2 = 512
CH3 = 1024


def _sc_segment_sum2(p, src2d, dst2d, zeros, n_pad, ch):
  F = p.shape[1]
  e_tile = src2d.shape[1]
  nfull = e_tile // ch
  tail = e_tile - nfull * ch
  rows_per_tile = n_pad // NS
  mesh = plsc.VectorSubcoreMesh(
      core_axis_name="c", subcore_axis_name="s", num_cores=NC,
      num_subcores=NS)

  @functools.partial(
      pl.kernel,
      out_type=jax.ShapeDtypeStruct((2 * n_pad, F), jnp.float32),
      mesh=mesh,
      compiler_params=pltpu.CompilerParams(use_tc_tiling_on_sc=False),
      scratch_types=[
          pltpu.VMEM((e_tile,), jnp.int32),
          pltpu.VMEM((e_tile,), jnp.int32),
          pltpu.VMEM((ch, F), jnp.float32),
          pltpu.VMEM_SHARED((n_pad, F), jnp.float32),
          pltpu.SemaphoreType.DMA,
      ],
  )
  def seg_kernel(p_hbm, src_hbm, dst_hbm, z_hbm, out_hbm,
                 src_v, dst_v, buf, agg_sh, sem):
    c = lax.axis_index("c")
    s = lax.axis_index("s")
    wid = c * NS + s
    r0 = s * rows_per_tile
    pltpu.sync_copy(src_hbm.at[wid], src_v)
    pltpu.sync_copy(dst_hbm.at[wid], dst_v)
    pltpu.sync_copy(z_hbm.at[pl.ds(r0, rows_per_tile)],
                    agg_sh.at[pl.ds(r0, rows_per_tile)])
    plsc.subcore_barrier()

    def body(j, carry):
      pltpu.async_copy(p_hbm.at[src_v.at[pl.ds(j * ch, ch)]], buf, sem).wait()
      pltpu.sync_copy(buf, agg_sh.at[dst_v.at[pl.ds(j * ch, ch)]], add=True)
      return carry

    lax.fori_loop(0, nfull, body, 0)
    if tail:
      o = nfull * ch
      pltpu.async_copy(p_hbm.at[src_v.at[pl.ds(o, tail)]],
                       buf.at[pl.ds(0, tail)], sem).wait()
      pltpu.sync_copy(buf.at[pl.ds(0, tail)],
                      agg_sh.at[dst_v.at[pl.ds(o, tail)]], add=True)
    plsc.subcore_barrier()
    pltpu.sync_copy(agg_sh.at[pl.ds(r0, rows_per_tile)],
                    out_hbm.at[pl.ds(c * n_pad + r0, rows_per_tile)])

  return seg_kernel(p, src2d, dst2d, zeros)


def _project1(x, wlT, wrT, n_pad, blk):
  D = x.shape[1]
  F1 = wlT.shape[1]
  grid = n_pad // blk

  def body(x_ref, wl_ref, wr_ref, p_ref, q_ref):
    xb = x_ref[...]
    pb = jnp.dot(xb, wl_ref[...], preferred_element_type=jnp.float32)
    q_ref[...] = jnp.dot(xb, wr_ref[...], preferred_element_type=jnp.float32)
    ones = jnp.ones((blk, 1), jnp.float32)
    zer = jnp.zeros((blk, 15), jnp.float32)
    p_ref[...] = jnp.concatenate([pb, ones, zer], axis=1)

  return pl.pallas_call(
      body,
      grid=(grid,),
      in_specs=[
          pl.BlockSpec((blk, D), lambda i: (i, 0)),
          pl.BlockSpec((D, F1), lambda i: (0, 0)),
          pl.BlockSpec((D, F1), lambda i: (0, 0)),
      ],
      out_specs=[
          pl.BlockSpec((blk, F1 + 16), lambda i: (i, 0)),
          pl.BlockSpec((blk, F1), lambda i: (i, 0)),
      ],
      out_shape=[
          jax.ShapeDtypeStruct((n_pad, F1 + 16), jnp.float32),
          jax.ShapeDtypeStruct((n_pad, F1), jnp.float32),
      ],
  )(x, wlT, wrT)


def _combine1(parts, q1, b1, wlT2, wrT2, n_pad, blk):
  Fw = parts.shape[1]
  F1 = Fw - 16
  F2 = wlT2.shape[1]
  grid = n_pad // blk

  def body(pa_ref, pb_ref, q_ref, b_ref, wl_ref, wr_ref,
           p2_ref, q2_ref, inv_ref):
    ps = pa_ref[...] + pb_ref[...]
    cnt = ps[:, F1:F1 + 1]
    inv = 1.0 / jnp.maximum(cnt, 1.0)
    o = ps[:, :F1] * inv + b_ref[...] + q_ref[...]
    nrm = jnp.sqrt(jnp.sum(o * o, axis=1, keepdims=True))
    h = o / jnp.maximum(nrm, 1e-12)
    h = jnp.maximum(h, 0.0)
    p2_ref[...] = jnp.dot(h, wl_ref[...], preferred_element_type=jnp.float32)
    q2_ref[...] = jnp.dot(h, wr_ref[...], preferred_element_type=jnp.float32)
    inv_ref[...] = inv

  return pl.pallas_call(
      body,
      grid=(grid,),
      in_specs=[
          pl.BlockSpec((blk, Fw), lambda i: (i, 0)),
          pl.BlockSpec((blk, Fw), lambda i: (i + grid, 0)),
          pl.BlockSpec((blk, F1), lambda i: (i, 0)),
          pl.BlockSpec((1, F1), lambda i: (0, 0)),
          pl.BlockSpec((F1, F2), lambda i: (0, 0)),
          pl.BlockSpec((F1, F2), lambda i: (0, 0)),
      ],
      out_specs=[
          pl.BlockSpec((blk, F2), lambda i: (i, 0)),
          pl.BlockSpec((blk, F2), lambda i: (i, 0)),
          pl.BlockSpec((blk, 1), lambda i: (i, 0)),
      ],
      out_shape=[
          jax.ShapeDtypeStruct((n_pad, F2), jnp.float32),
          jax.ShapeDtypeStruct((n_pad, F2), jnp.float32),
          jax.ShapeDtypeStruct((n_pad, 1), jnp.float32),
      ],
  )(parts, parts, q1, b1[None, :], wlT2, wrT2)


def _combine2(parts, q2, b2, inv, wlT3, wrT3, n_pad, blk):
  F2 = parts.shape[1]
  F3 = wlT3.shape[1]
  grid = n_pad // blk

  def body(pa_ref, pb_ref, q_ref, b_ref, inv_ref, wl_ref, wr_ref,
           p3_ref, q3_ref):
    ps = pa_ref[...] + pb_ref[...]
    o = ps * inv_ref[...] + b_ref[...] + q_ref[...]
    nrm = jnp.sqrt(jnp.sum(o * o, axis=1, keepdims=True))
    h = o / jnp.maximum(nrm, 1e-12)
    h = jnp.maximum(h, 0.0)
    p3_ref[...] = jnp.dot(h, wl_ref[...], preferred_element_type=jnp.float32)
    q3_ref[...] = jnp.dot(h, wr_ref[...], preferred_element_type=jnp.float32)

  return pl.pallas_call(
      body,
      grid=(grid,),
      in_specs=[
          pl.BlockSpec((blk, F2), lambda i: (i, 0)),
          pl.BlockSpec((blk, F2), lambda i: (i + grid, 0)),
          pl.BlockSpec((blk, F2), lambda i: (i, 0)),
          pl.BlockSpec((1, F2), lambda i: (0, 0)),
          pl.BlockSpec((blk, 1), lambda i: (i, 0)),
          pl.BlockSpec((F2, F3), lambda i: (0, 0)),
          pl.BlockSpec((F2, F3), lambda i: (0, 0)),
      ],
      out_specs=[
          pl.BlockSpec((blk, F3), lambda i: (i, 0)),
          pl.BlockSpec((blk, F3), lambda i: (i, 0)),
      ],
      out_shape=[
          jax.ShapeDtypeStruct((n_pad, F3), jnp.float32),
          jax.ShapeDtypeStruct((n_pad, F3), jnp.float32),
      ],
  )(parts, parts, q2, b2[None, :], inv, wlT3, wrT3)


def _final(parts, q3, b3, inv, watt, wfcT, bfc, wsT, bs, n, n_pad):
  F3 = watt.shape[0]
  HID = wfcT.shape[1]

  def body(pa_ref, pb_ref, q_ref, b_ref, inv_ref, wa_ref, wfc_ref, bfc_ref,
           ws_ref, bs_ref, out_ref):
    ps = pa_ref[...] + pb_ref[...]
    o = ps * inv_ref[...] + b_ref[...] + q_ref[...]
    nrm = jnp.sqrt(jnp.sum(o * o, axis=1, keepdims=True))
    h = o / jnp.maximum(nrm, 1e-12)
    cs = jnp.sum(h, axis=0, keepdims=True) / n
    gc = jnp.dot(cs, wa_ref[...], preferred_element_type=jnp.float32)
    tg = jnp.tanh(gc)
    scores = jax.nn.sigmoid(jnp.sum(h * tg, axis=1, keepdims=True))
    rep = jnp.sum(h * scores, axis=0, keepdims=True)
    s1 = jnp.dot(rep, wfc_ref[...], preferred_element_type=jnp.float32)
    s1 = jnp.maximum(s1 + bfc_ref[...], 0.0)
    s2 = jnp.dot(s1, ws_ref[...], preferred_element_type=jnp.float32)
    out_ref[...] = jax.nn.sigmoid(s2 + bs_ref[...])

  return pl.pallas_call(
      body,
      in_specs=[
          pl.BlockSpec((n, F3), lambda: (0, 0)),
          pl.BlockSpec((n, F3), lambda: (0, 0)),
          pl.BlockSpec((n, F3), lambda: (0, 0)),
          pl.BlockSpec((1, F3), lambda: (0, 0)),
          pl.BlockSpec((n, 1), lambda: (0, 0)),
          pl.BlockSpec((F3, F3), lambda: (0, 0)),
          pl.BlockSpec((F3, HID), lambda: (0, 0)),
          pl.BlockSpec((1, HID), lambda: (0, 0)),
          pl.BlockSpec((HID, 1), lambda: (0, 0)),
          pl.BlockSpec((1, 1), lambda: (0, 0)),
      ],
      out_specs=pl.BlockSpec((1, 1), lambda: (0, 0)),
      out_shape=jax.ShapeDtypeStruct((1, 1), jnp.float32),
  )(parts[:n], parts[n_pad:n_pad + n], q3[:n], b3[None, :], inv[:n],
    watt, wfcT, bfc[None, :], wsT, bs[None, :])


def kernel(features_1, edge_index_1, W_l1, b_l1, W_r1, W_l2, b_l2, W_r2,
           W_l3, b_l3, W_r3, W_att, W_fc, b_fc, W_s, b_s):
  n, d = features_1.shape
  e = edge_index_1.shape[1]

  n_pad = ((n + NS * 8 - 1) // (NS * 8)) * (NS * 8)
  e_tile = ((e + NW * CH - 1) // (NW * CH)) * CH
  e_pad = e_tile * NW

  f1 = W_l1.shape[0]
  f2 = W_l2.shape[0]
  f3 = W_l3.shape[0]

  x = jnp.zeros((n_pad, d), jnp.float32).at[:n].set(features_1)
  src = jnp.concatenate(
      [edge_index_1[0].astype(jnp.int32),
       jnp.zeros((e_pad - e,), jnp.int32)]).reshape(NW, e_tile)
  pad_dst = n + jnp.arange(e_pad - e, dtype=jnp.int32) % (n_pad - n)
  dst = jnp.concatenate(
      [edge_index_1[1].astype(jnp.int32), pad_dst]).reshape(NW, e_tile)
  del pad_dst

  z1 = jnp.zeros((n_pad, f1 + 16), jnp.float32)
  z2 = jnp.zeros((n_pad, f2), jnp.float32)
  z3 = jnp.zeros((n_pad, f3), jnp.float32)

  blk = n_pad // 8

  p1, q1 = _project1(x, W_l1.T, W_r1.T, n_pad, blk)
  parts1 = _sc_segment_sum2(p1, src, dst, z1, n_pad, CH)
  p2, q2, inv = _combine1(parts1, q1, b_l1, W_l2.T, W_r2.T, n_pad, blk)
  parts2 = _sc_segment_sum2(p2, src, dst, z2, n_pad, CH2)
  p3, q3 = _combine2(parts2, q2, b_l2, inv, W_l3.T, W_r3.T, n_pad, blk)
  parts3 = _sc_segment_sum2(p3, src, dst, z3, n_pad, CH3)
  return _final(parts3, q3, b_l3, inv, W_att, W_fc.T, b_fc, W_s.T, b_s,
                n, n_pad)


# speedup vs baseline: 1.4059x; 1.0006x over previous
import functools

import jax
import jax.numpy as jnp
from jax import lax
from jax.experimental import pallas as pl
from jax.experimental.pallas import tpu as pltpu
from jax.experimental.pallas import tpu_sc as plsc

NC = 2
NS = 16
NW = NC * NS
CH = 128
CH2 = 512
CH3 = 1024


def _sc_segment_sum(p, src2d, dst2d, zeros, n_pad, ch):
  F = p.shape[1]
  e_tile = src2d.shape[1]
  nfull = e_tile // ch
  tail = e_tile - nfull * ch
  rows_per_tile = n_pad // NS
  mesh = plsc.VectorSubcoreMesh(
      core_axis_name="c", subcore_axis_name="s", num_cores=NC,
      num_subcores=NS)

  @functools.partial(
      pl.kernel,
      out_type=jax.ShapeDtypeStruct((2 * n_pad, F), jnp.float32),
      mesh=mesh,
      compiler_params=pltpu.CompilerParams(use_tc_tiling_on_sc=False),
      scratch_types=[
          pltpu.VMEM((e_tile,), jnp.int32),
          pltpu.VMEM((e_tile,), jnp.int32),
          pltpu.VMEM((ch, F), jnp.float32),
          pltpu.VMEM_SHARED((n_pad, F), jnp.float32),
          pltpu.SemaphoreType.DMA,
      ],
  )
  def seg_kernel(p_hbm, src_hbm, dst_hbm, z_hbm, out_hbm,
                 src_v, dst_v, buf, agg_sh, sem):
    c = lax.axis_index("c")
    s = lax.axis_index("s")
    wid = c * NS + s
    r0 = s * rows_per_tile
    pltpu.sync_copy(src_hbm.at[wid], src_v)
    pltpu.sync_copy(dst_hbm.at[wid], dst_v)
    pltpu.sync_copy(z_hbm.at[pl.ds(r0, rows_per_tile)],
                    agg_sh.at[pl.ds(r0, rows_per_tile)])
    plsc.subcore_barrier()

    def body(j, carry):
      pltpu.async_copy(p_hbm.at[src_v.at[pl.ds(j * ch, ch)]], buf, sem).wait()
      pltpu.sync_copy(buf, agg_sh.at[dst_v.at[pl.ds(j * ch, ch)]], add=True)
      return carry

    lax.fori_loop(0, nfull, body, 0)
    if tail:
      o = nfull * ch
      pltpu.async_copy(p_hbm.at[src_v.at[pl.ds(o, tail)]],
                       buf.at[pl.ds(0, tail)], sem).wait()
      pltpu.sync_copy(buf.at[pl.ds(0, tail)],
                      agg_sh.at[dst_v.at[pl.ds(o, tail)]], add=True)
    plsc.subcore_barrier()
    pltpu.sync_copy(agg_sh.at[pl.ds(r0, rows_per_tile)],
                    out_hbm.at[pl.ds(c * n_pad + r0, rows_per_tile)])

  return seg_kernel(p, src2d, dst2d, zeros)


def _project1(x, wlT, wrT, n_pad, blk):
  D = x.shape[1]
  F1 = wlT.shape[1]
  grid = n_pad // blk

  def body(x_ref, wl_ref, wr_ref, p_ref, q_ref):
    xb = x_ref[...]
    pb = jnp.dot(xb, wl_ref[...], preferred_element_type=jnp.float32)
    q_ref[...] = jnp.dot(xb, wr_ref[...], preferred_element_type=jnp.float32)
    ones = jnp.ones((blk, 1), jnp.float32)
    zer = jnp.zeros((blk, 15), jnp.float32)
    p_ref[...] = jnp.concatenate([pb, ones, zer], axis=1)

  return pl.pallas_call(
      body,
      grid=(grid,),
      in_specs=[
          pl.BlockSpec((blk, D), lambda i: (i, 0)),
          pl.BlockSpec((D, F1), lambda i: (0, 0)),
          pl.BlockSpec((D, F1), lambda i: (0, 0)),
      ],
      out_specs=[
          pl.BlockSpec((blk, F1 + 16), lambda i: (i, 0)),
          pl.BlockSpec((blk, F1), lambda i: (i, 0)),
      ],
      out_shape=[
          jax.ShapeDtypeStruct((n_pad, F1 + 16), jnp.float32),
          jax.ShapeDtypeStruct((n_pad, F1), jnp.float32),
      ],
  )(x, wlT, wrT)


def _combine1(parts, q1, b1, wlT2, wrT2, n_pad, blk):
  Fw = parts.shape[1]
  F1 = Fw - 16
  F2 = wlT2.shape[1]
  grid = n_pad // blk

  def body(pa_ref, pb_ref, q_ref, b_ref, wl_ref, wr_ref,
           p2_ref, q2_ref, inv_ref):
    ps = pa_ref[...] + pb_ref[...]
    cnt = ps[:, F1:F1 + 1]
    inv = 1.0 / jnp.maximum(cnt, 1.0)
    o = ps[:, :F1] * inv + b_ref[...] + q_ref[...]
    nrm = jnp.sqrt(jnp.sum(o * o, axis=1, keepdims=True))
    h = o / jnp.maximum(nrm, 1e-12)
    h = jnp.maximum(h, 0.0)
    p2_ref[...] = jnp.dot(h, wl_ref[...], preferred_element_type=jnp.float32)
    q2_ref[...] = jnp.dot(h, wr_ref[...], preferred_element_type=jnp.float32)
    inv_ref[...] = inv

  return pl.pallas_call(
      body,
      grid=(grid,),
      in_specs=[
          pl.BlockSpec((blk, Fw), lambda i: (i, 0)),
          pl.BlockSpec((blk, Fw), lambda i: (i + grid, 0)),
          pl.BlockSpec((blk, F1), lambda i: (i, 0)),
          pl.BlockSpec((1, F1), lambda i: (0, 0)),
          pl.BlockSpec((F1, F2), lambda i: (0, 0)),
          pl.BlockSpec((F1, F2), lambda i: (0, 0)),
      ],
      out_specs=[
          pl.BlockSpec((blk, F2), lambda i: (i, 0)),
          pl.BlockSpec((blk, F2), lambda i: (i, 0)),
          pl.BlockSpec((blk, 1), lambda i: (i, 0)),
      ],
      out_shape=[
          jax.ShapeDtypeStruct((n_pad, F2), jnp.float32),
          jax.ShapeDtypeStruct((n_pad, F2), jnp.float32),
          jax.ShapeDtypeStruct((n_pad, 1), jnp.float32),
      ],
  )(parts, parts, q1, b1[None, :], wlT2, wrT2)


def _combine2(parts, q2, b2, inv, wlT3, wrT3, n_pad, blk):
  F2 = parts.shape[1]
  F3 = wlT3.shape[1]
  grid = n_pad // blk

  def body(pa_ref, pb_ref, q_ref, b_ref, inv_ref, wl_ref, wr_ref,
           p3_ref, q3_ref):
    ps = pa_ref[...] + pb_ref[...]
    o = ps * inv_ref[...] + b_ref[...] + q_ref[...]
    nrm = jnp.sqrt(jnp.sum(o * o, axis=1, keepdims=True))
    h = o / jnp.maximum(nrm, 1e-12)
    h = jnp.maximum(h, 0.0)
    p3_ref[...] = jnp.dot(h, wl_ref[...], preferred_element_type=jnp.float32)
    q3_ref[...] = jnp.dot(h, wr_ref[...], preferred_element_type=jnp.float32)

  return pl.pallas_call(
      body,
      grid=(grid,),
      in_specs=[
          pl.BlockSpec((blk, F2), lambda i: (i, 0)),
          pl.BlockSpec((blk, F2), lambda i: (i + grid, 0)),
          pl.BlockSpec((blk, F2), lambda i: (i, 0)),
          pl.BlockSpec((1, F2), lambda i: (0, 0)),
          pl.BlockSpec((blk, 1), lambda i: (i, 0)),
          pl.BlockSpec((F2, F3), lambda i: (0, 0)),
          pl.BlockSpec((F2, F3), lambda i: (0, 0)),
      ],
      out_specs=[
          pl.BlockSpec((blk, F3), lambda i: (i, 0)),
          pl.BlockSpec((blk, F3), lambda i: (i, 0)),
      ],
      out_shape=[
          jax.ShapeDtypeStruct((n_pad, F3), jnp.float32),
          jax.ShapeDtypeStruct((n_pad, F3), jnp.float32),
      ],
  )(parts, parts, q2, b2[None, :], inv, wlT3, wrT3)


def _final(parts, q3, b3, inv, watt, wfcT, bfc, wsT, bs, n, n_pad):
  F3 = watt.shape[0]
  HID = wfcT.shape[1]

  def body(pa_ref, pb_ref, q_ref, b_ref, inv_ref, wa_ref, wfc_ref, bfc_ref,
           ws_ref, bs_ref, out_ref):
    ps = pa_ref[...] + pb_ref[...]
    o = ps * inv_ref[...] + b_ref[...] + q_ref[...]
    nrm = jnp.sqrt(jnp.sum(o * o, axis=1, keepdims=True))
    h = o / jnp.maximum(nrm, 1e-12)
    cs = jnp.sum(h, axis=0, keepdims=True) / n
    gc = jnp.dot(cs, wa_ref[...], preferred_element_type=jnp.float32)
    tg = jnp.tanh(gc)
    scores = jax.nn.sigmoid(jnp.sum(h * tg, axis=1, keepdims=True))
    rep = jnp.sum(h * scores, axis=0, keepdims=True)
    s1 = jnp.dot(rep, wfc_ref[...], preferred_element_type=jnp.float32)
    s1 = jnp.maximum(s1 + bfc_ref[...], 0.0)
    s2 = jnp.dot(s1, ws_ref[...], preferred_element_type=jnp.float32)
    out_ref[...] = jax.nn.sigmoid(s2 + bs_ref[...])

  return pl.pallas_call(
      body,
      in_specs=[
          pl.BlockSpec((n, F3), lambda: (0, 0)),
          pl.BlockSpec((n, F3), lambda: (0, 0)),
          pl.BlockSpec((n, F3), lambda: (0, 0)),
          pl.BlockSpec((1, F3), lambda: (0, 0)),
          pl.BlockSpec((n, 1), lambda: (0, 0)),
          pl.BlockSpec((F3, F3), lambda: (0, 0)),
          pl.BlockSpec((F3, HID), lambda: (0, 0)),
          pl.BlockSpec((1, HID), lambda: (0, 0)),
          pl.BlockSpec((HID, 1), lambda: (0, 0)),
          pl.BlockSpec((1, 1), lambda: (0, 0)),
      ],
      out_specs=pl.BlockSpec((1, 1), lambda: (0, 0)),
      out_shape=jax.ShapeDtypeStruct((1, 1), jnp.float32),
  )(parts[:n], parts[n_pad:n_pad + n], q3[:n], b3[None, :], inv[:n],
    watt, wfcT, bfc[None, :], wsT, bs[None, :])


def kernel(features_1, edge_index_1, W_l1, b_l1, W_r1, W_l2, b_l2, W_r2,
           W_l3, b_l3, W_r3, W_att, W_fc, b_fc, W_s, b_s):
  n, d = features_1.shape
  e = edge_index_1.shape[1]

  n_pad = ((n + NS * 8 - 1) // (NS * 8)) * (NS * 8)
  e_tile = ((e + NW * CH - 1) // (NW * CH)) * CH
  e_pad = e_tile * NW

  f1 = W_l1.shape[0]
  f2 = W_l2.shape[0]
  f3 = W_l3.shape[0]

  x = jnp.zeros((n_pad, d), jnp.float32).at[:n].set(features_1)
  src = jnp.concatenate(
      [edge_index_1[0].astype(jnp.int32),
       jnp.zeros((e_pad - e,), jnp.int32)]).reshape(NW, e_tile)
  pad_dst = n + jnp.arange(e_pad - e, dtype=jnp.int32) % (n_pad - n)
  dst = jnp.concatenate(
      [edge_index_1[1].astype(jnp.int32), pad_dst]).reshape(NW, e_tile)
  del pad_dst

  z1 = jnp.zeros((n_pad, f1 + 16), jnp.float32)
  z2 = jnp.zeros((n_pad, f2), jnp.float32)
  z3 = jnp.zeros((n_pad, f3), jnp.float32)

  blk = n_pad // 8

  p1, q1 = _project1(x, W_l1.T, W_r1.T, n_pad, blk)
  parts1 = _sc_segment_sum(p1, src, dst, z1, n_pad, CH)
  p2, q2, inv = _combine1(parts1, q1, b_l1, W_l2.T, W_r2.T, n_pad, blk)
  parts2 = _sc_segment_sum(p2, src, dst, z2, n_pad, CH2)
  p3, q3 = _combine2(parts2, q2, b_l2, inv, W_l3.T, W_r3.T, n_pad, blk)
  parts3 = _sc_segment_sum(p3, src, dst, z3, n_pad, CH3)
  return _final(parts3, q3, b_l3, inv, W_att, W_fc.T, b_fc, W_s.T, b_s,
                n, n_pad)
